# baseline (speedup 1.0000x reference)
import functools

import jax
import jax.numpy as jnp
from jax import lax
from jax.experimental import pallas as pl
from jax.experimental.pallas import tpu as pltpu
from jax.experimental.pallas import tpu_sc as plsc

B = 16384
D = 128
NC = 2
NS = 16
L = 16
NW = NC * NS
BPW = B // NW
CH = 128
NCH = BPW // CH
UD = 16
NACC = 4


def _sc_body(idx_hbm, ent_hbm, path_hbm, out_hbm,
             idx_raw_v, idx_path_v, idx_pos_v, idx_neg_v,
             pos0, neg0, path0, pos1, neg1, path1, out_v,
             sa0, sa1, sa2, sb0, sb1, sb2):
    w = lax.axis_index("s") * NC + lax.axis_index("c")
    base = w * BPW
    pltpu.sync_copy(idx_hbm.at[pl.ds(base, BPW)], idx_raw_v)
    lane = lax.iota(jnp.int32, L)
    zero16 = lane * 0

    def deint_body(i, _):
        r0 = lane + i * L
        idx_path_v[pl.ds(i * L, L)] = plsc.load_gather(idx_raw_v, [r0, zero16])
        idx_pos_v[pl.ds(i * L, L)] = plsc.load_gather(idx_raw_v, [r0, zero16 + 1])
        idx_neg_v[pl.ds(i * L, L)] = plsc.load_gather(idx_raw_v, [r0, zero16 + 2])
        return 0

    lax.fori_loop(0, BPW // L, deint_body, 0)

    bufs = ((pos0, neg0, path0, sa0, sa1, sa2),
            (pos1, neg1, path1, sb0, sb1, sb2))

    def issue(c, pos_b, neg_b, path_b, s0, s1, s2):
        sl = pl.ds(c * CH, CH)
        return (pltpu.async_copy(ent_hbm.at[idx_pos_v.at[sl]], pos_b, s0),
                pltpu.async_copy(ent_hbm.at[idx_neg_v.at[sl]], neg_b, s1),
                pltpu.async_copy(path_hbm.at[idx_path_v.at[sl]], path_b, s2))

    def compute(c, pos_b, neg_b, path_b, *_):
        lane = lax.iota(jnp.int32, L)

        def group_body(g, _):
            rows = lane + g * L

            def block_body(bb, accs):
                d0 = bb * UD
                accs = list(accs)
                for k in range(UD):
                    dsp = (lane + (d0 + k)) & (D - 1)
                    p = plsc.load_gather(pos_b, [rows, dsp])
                    n = plsc.load_gather(neg_b, [rows, dsp])
                    t = plsc.load_gather(path_b, [rows, dsp])
                    accs[k % NACC] = accs[k % NACC] + (p - n) * t
                return tuple(accs)

            zero = jnp.zeros((L,), jnp.float32)
            accs = lax.fori_loop(0, D // UD, block_body, (zero,) * NACC)
            acc = (accs[0] + accs[1]) + (accs[2] + accs[3])
            out_v[pl.ds(c * CH + g * L, L)] = acc
            return 0

        lax.fori_loop(0, CH // L, group_body, 0)

    pending = {0: issue(0, *bufs[0])}
    for c in range(NCH):
        if c + 1 < NCH:
            pending[c + 1] = issue(c + 1, *bufs[(c + 1) % 2])
        for desc in pending.pop(c):
            desc.wait()
        compute(c, *bufs[c % 2])

    pltpu.sync_copy(out_v, out_hbm.at[pl.ds(base, BPW)])


@functools.partial(
    pl.kernel,
    out_type=jax.ShapeDtypeStruct((B,), jnp.float32),
    mesh=plsc.VectorSubcoreMesh(core_axis_name="c", subcore_axis_name="s"),
    compiler_params=pltpu.CompilerParams(
        needs_layout_passes=False, disable_bounds_checks=True,
        use_tc_tiling_on_sc=False),
    scratch_types=[
        pltpu.VMEM((BPW, 3), jnp.int32),
        pltpu.VMEM((BPW,), jnp.int32),
        pltpu.VMEM((BPW,), jnp.int32),
        pltpu.VMEM((BPW,), jnp.int32),
        pltpu.VMEM((CH, D), jnp.float32),
        pltpu.VMEM((CH, D), jnp.float32),
        pltpu.VMEM((CH, D), jnp.float32),
        pltpu.VMEM((CH, D), jnp.float32),
        pltpu.VMEM((CH, D), jnp.float32),
        pltpu.VMEM((CH, D), jnp.float32),
        pltpu.VMEM((BPW,), jnp.float32),
        pltpu.SemaphoreType.DMA,
        pltpu.SemaphoreType.DMA,
        pltpu.SemaphoreType.DMA,
        pltpu.SemaphoreType.DMA,
        pltpu.SemaphoreType.DMA,
        pltpu.SemaphoreType.DMA,
    ],
)
def _uni_model_sc(idx_hbm, ent_hbm, path_hbm, out_hbm, *rest):
    _sc_body(idx_hbm, ent_hbm, path_hbm, out_hbm, *rest)


@jax.jit
def kernel(ents_path_idxs, ent_table, path_table):
    out = _uni_model_sc(ents_path_idxs.astype(jnp.int32), ent_table, path_table)
    return out.reshape(B, 1, 1)


# speedup vs baseline: 1.4196x; 1.4196x over previous
import functools

import jax
import jax.numpy as jnp
from jax import lax
from jax.experimental import pallas as pl
from jax.experimental.pallas import tpu as pltpu
from jax.experimental.pallas import tpu_sc as plsc

B = 16384
D = 128
NC = 2
NS = 16
L = 16
NW = NC * NS
BPW = B // NW
CH = 128
NCH = BPW // CH
UD = 16
NACC = 4


def _sc_body(idx_hbm, ent_hbm, path_hbm, out_hbm,
             idx_path_v, idx_pos_v, idx_neg_v,
             pos0, neg0, path0, pos1, neg1, path1, out_v,
             sa0, sa1, sa2, sb0, sb1, sb2):
    w = lax.axis_index("s") * NC + lax.axis_index("c")
    base = w * BPW
    pltpu.sync_copy(idx_hbm.at[0, w], idx_path_v)
    pltpu.sync_copy(idx_hbm.at[1, w], idx_pos_v)
    pltpu.sync_copy(idx_hbm.at[2, w], idx_neg_v)

    bufs = ((pos0, neg0, path0, sa0, sa1, sa2),
            (pos1, neg1, path1, sb0, sb1, sb2))

    def issue(c, pos_b, neg_b, path_b, s0, s1, s2):
        return (pltpu.async_copy(ent_hbm.at[idx_pos_v.at[c]], pos_b, s0),
                pltpu.async_copy(ent_hbm.at[idx_neg_v.at[c]], neg_b, s1),
                pltpu.async_copy(path_hbm.at[idx_path_v.at[c]], path_b, s2))

    def compute(c, pos_b, neg_b, path_b, *_):
        lane = lax.iota(jnp.int32, L)

        def group_body(g, _):
            rows = lane + g * L

            def block_body(bb, accs):
                d0 = bb * UD
                accs = list(accs)
                for k in range(UD):
                    dsp = (lane + (d0 + k)) & (D - 1)
                    p = plsc.load_gather(pos_b, [rows, dsp])
                    n = plsc.load_gather(neg_b, [rows, dsp])
                    t = plsc.load_gather(path_b, [rows, dsp])
                    accs[k % NACC] = accs[k % NACC] + (p - n) * t
                return tuple(accs)

            zero = jnp.zeros((L,), jnp.float32)
            accs = lax.fori_loop(0, D // UD, block_body, (zero,) * NACC)
            acc = (accs[0] + accs[1]) + (accs[2] + accs[3])
            out_v[pl.ds(c * CH + g * L, L)] = acc
            return 0

        lax.fori_loop(0, CH // L, group_body, 0)

    pending = {0: issue(0, *bufs[0])}
    for c in range(NCH):
        if c + 1 < NCH:
            pending[c + 1] = issue(c + 1, *bufs[(c + 1) % 2])
        for desc in pending.pop(c):
            desc.wait()
        compute(c, *bufs[c % 2])

    pltpu.sync_copy(out_v, out_hbm.at[pl.ds(base, BPW)])


@functools.partial(
    pl.kernel,
    out_type=jax.ShapeDtypeStruct((B,), jnp.float32),
    mesh=plsc.VectorSubcoreMesh(core_axis_name="c", subcore_axis_name="s"),
    compiler_params=pltpu.CompilerParams(
        needs_layout_passes=False, disable_bounds_checks=True),
    scratch_types=[
        pltpu.VMEM((NCH, CH), jnp.int32),
        pltpu.VMEM((NCH, CH), jnp.int32),
        pltpu.VMEM((NCH, CH), jnp.int32),
        pltpu.VMEM((CH, D), jnp.float32),
        pltpu.VMEM((CH, D), jnp.float32),
        pltpu.VMEM((CH, D), jnp.float32),
        pltpu.VMEM((CH, D), jnp.float32),
        pltpu.VMEM((CH, D), jnp.float32),
        pltpu.VMEM((CH, D), jnp.float32),
        pltpu.VMEM((BPW,), jnp.float32),
        pltpu.SemaphoreType.DMA,
        pltpu.SemaphoreType.DMA,
        pltpu.SemaphoreType.DMA,
        pltpu.SemaphoreType.DMA,
        pltpu.SemaphoreType.DMA,
        pltpu.SemaphoreType.DMA,
    ],
)
def _uni_model_sc(idx_hbm, ent_hbm, path_hbm, out_hbm, *rest):
    _sc_body(idx_hbm, ent_hbm, path_hbm, out_hbm, *rest)


@jax.jit
def kernel(ents_path_idxs, ent_table, path_table):
    idxs = ents_path_idxs.astype(jnp.int32).T.reshape(3, NW, NCH, CH)
    out = _uni_model_sc(idxs, ent_table, path_table)
    return out.reshape(B, 1, 1)


# speedup vs baseline: 1.4435x; 1.0169x over previous
import functools

import jax
import jax.numpy as jnp
from jax import lax
from jax.experimental import pallas as pl
from jax.experimental.pallas import tpu as pltpu
from jax.experimental.pallas import tpu_sc as plsc

B = 16384
D = 128
NC = 2
NS = 16
L = 16
NW = NC * NS
BPW = B // NW
CH = 128
NCH = BPW // CH
UD = 8
NACC = 4


def _sc_body(idx_hbm, ent_hbm, path_hbm, out_hbm,
             idx_path_v, idx_pos_v, idx_neg_v,
             pos0, neg0, path0, pos1, neg1, path1, out_v,
             sa0, sa1, sa2, sb0, sb1, sb2):
    w = lax.axis_index("s") * NC + lax.axis_index("c")
    base = w * BPW
    pltpu.sync_copy(idx_hbm.at[0, w], idx_path_v)
    pltpu.sync_copy(idx_hbm.at[1, w], idx_pos_v)
    pltpu.sync_copy(idx_hbm.at[2, w], idx_neg_v)

    bufs = ((pos0, neg0, path0, sa0, sa1, sa2),
            (pos1, neg1, path1, sb0, sb1, sb2))

    def issue(c, pos_b, neg_b, path_b, s0, s1, s2):
        return (pltpu.async_copy(ent_hbm.at[idx_pos_v.at[c]], pos_b, s0),
                pltpu.async_copy(ent_hbm.at[idx_neg_v.at[c]], neg_b, s1),
                pltpu.async_copy(path_hbm.at[idx_path_v.at[c]], path_b, s2))

    def compute(c, pos_b, neg_b, path_b, *_):
        lane = lax.iota(jnp.int32, L)

        def group_body(g, _):
            rows = lane + g * L

            def block_body(bb, accs):
                d0 = bb * UD
                accs = list(accs)
                for k in range(UD):
                    dsp = (lane + (d0 + k)) & (D - 1)
                    p = plsc.load_gather(pos_b, [rows, dsp])
                    n = plsc.load_gather(neg_b, [rows, dsp])
                    t = plsc.load_gather(path_b, [rows, dsp])
                    accs[k % NACC] = accs[k % NACC] + (p - n) * t
                return tuple(accs)

            zero = jnp.zeros((L,), jnp.float32)
            accs = lax.fori_loop(0, D // UD, block_body, (zero,) * NACC)
            acc = (accs[0] + accs[1]) + (accs[2] + accs[3])
            out_v[pl.ds(c * CH + g * L, L)] = acc
            return 0

        lax.fori_loop(0, CH // L, group_body, 0)

    pending = {0: issue(0, *bufs[0])}
    for c in range(NCH):
        if c + 1 < NCH:
            pending[c + 1] = issue(c + 1, *bufs[(c + 1) % 2])
        for desc in pending.pop(c):
            desc.wait()
        compute(c, *bufs[c % 2])

    pltpu.sync_copy(out_v, out_hbm.at[pl.ds(base, BPW)])


@functools.partial(
    pl.kernel,
    out_type=jax.ShapeDtypeStruct((B,), jnp.float32),
    mesh=plsc.VectorSubcoreMesh(core_axis_name="c", subcore_axis_name="s"),
    compiler_params=pltpu.CompilerParams(
        needs_layout_passes=False, disable_bounds_checks=True),
    scratch_types=[
        pltpu.VMEM((NCH, CH), jnp.int32),
        pltpu.VMEM((NCH, CH), jnp.int32),
        pltpu.VMEM((NCH, CH), jnp.int32),
        pltpu.VMEM((CH, D), jnp.float32),
        pltpu.VMEM((CH, D), jnp.float32),
        pltpu.VMEM((CH, D), jnp.float32),
        pltpu.VMEM((CH, D), jnp.float32),
        pltpu.VMEM((CH, D), jnp.float32),
        pltpu.VMEM((CH, D), jnp.float32),
        pltpu.VMEM((BPW,), jnp.float32),
        pltpu.SemaphoreType.DMA,
        pltpu.SemaphoreType.DMA,
        pltpu.SemaphoreType.DMA,
        pltpu.SemaphoreType.DMA,
        pltpu.SemaphoreType.DMA,
        pltpu.SemaphoreType.DMA,
    ],
)
def _uni_model_sc(idx_hbm, ent_hbm, path_hbm, out_hbm, *rest):
    _sc_body(idx_hbm, ent_hbm, path_hbm, out_hbm, *rest)


@jax.jit
def kernel(ents_path_idxs, ent_table, path_table):
    idxs = ents_path_idxs.astype(jnp.int32).T.reshape(3, NW, NCH, CH)
    out = _uni_model_sc(idxs, ent_table, path_table)
    return out.reshape(B, 1, 1)


# speedup vs baseline: 1.4842x; 1.0282x over previous
import functools

import jax
import jax.numpy as jnp
from jax import lax
from jax.experimental import pallas as pl
from jax.experimental.pallas import tpu as pltpu
from jax.experimental.pallas import tpu_sc as plsc

B = 16384
D = 128
NC = 2
NS = 16
L = 16
NW = NC * NS
BPW = B // NW
CH = 128
NCH = BPW // CH
UD = 8
NACC = 4


def _sc_body(idx_hbm, ent_hbm, path_hbm, out_hbm,
             idx_v,
             pos0, neg0, path0, pos1, neg1, path1, out_v,
             sa0, sa1, sa2, sb0, sb1, sb2):
    w = lax.axis_index("s") * NC + lax.axis_index("c")
    base = w * BPW
    pltpu.sync_copy(idx_hbm.at[w], idx_v)

    bufs = ((pos0, neg0, path0, sa0, sa1, sa2),
            (pos1, neg1, path1, sb0, sb1, sb2))

    def issue(c, pos_b, neg_b, path_b, s0, s1, s2):
        return (pltpu.async_copy(ent_hbm.at[idx_v.at[1, c]], pos_b, s0),
                pltpu.async_copy(ent_hbm.at[idx_v.at[2, c]], neg_b, s1),
                pltpu.async_copy(path_hbm.at[idx_v.at[0, c]], path_b, s2))

    def compute(c, pos_b, neg_b, path_b, *_):
        lane = lax.iota(jnp.int32, L)

        def group_body(g, _):
            rows = lane + g * L

            def block_body(bb, accs):
                d0 = bb * UD
                accs = list(accs)
                for k in range(UD):
                    dsp = (lane + (d0 + k)) & (D - 1)
                    p = plsc.load_gather(pos_b, [rows, dsp])
                    n = plsc.load_gather(neg_b, [rows, dsp])
                    t = plsc.load_gather(path_b, [rows, dsp])
                    accs[k % NACC] = accs[k % NACC] + (p - n) * t
                return tuple(accs)

            zero = jnp.zeros((L,), jnp.float32)
            accs = lax.fori_loop(0, D // UD, block_body, (zero,) * NACC)
            acc = (accs[0] + accs[1]) + (accs[2] + accs[3])
            out_v[pl.ds(c * CH + g * L, L)] = acc
            return 0

        lax.fori_loop(0, CH // L, group_body, 0)

    pending = {0: issue(0, *bufs[0])}
    for c in range(NCH):
        if c + 1 < NCH:
            pending[c + 1] = issue(c + 1, *bufs[(c + 1) % 2])
        for desc in pending.pop(c):
            desc.wait()
        compute(c, *bufs[c % 2])

    pltpu.sync_copy(out_v, out_hbm.at[pl.ds(base, BPW)])


@functools.partial(
    pl.kernel,
    out_type=jax.ShapeDtypeStruct((B,), jnp.float32),
    mesh=plsc.VectorSubcoreMesh(core_axis_name="c", subcore_axis_name="s"),
    compiler_params=pltpu.CompilerParams(
        needs_layout_passes=False, disable_bounds_checks=True),
    scratch_types=[
        pltpu.VMEM((3, NCH, CH), jnp.int32),
        pltpu.VMEM((CH, D), jnp.float32),
        pltpu.VMEM((CH, D), jnp.float32),
        pltpu.VMEM((CH, D), jnp.float32),
        pltpu.VMEM((CH, D), jnp.float32),
        pltpu.VMEM((CH, D), jnp.float32),
        pltpu.VMEM((CH, D), jnp.float32),
        pltpu.VMEM((BPW,), jnp.float32),
        pltpu.SemaphoreType.DMA,
        pltpu.SemaphoreType.DMA,
        pltpu.SemaphoreType.DMA,
        pltpu.SemaphoreType.DMA,
        pltpu.SemaphoreType.DMA,
        pltpu.SemaphoreType.DMA,
    ],
)
def _uni_model_sc(idx_hbm, ent_hbm, path_hbm, out_hbm, *rest):
    _sc_body(idx_hbm, ent_hbm, path_hbm, out_hbm, *rest)


@jax.jit
def kernel(ents_path_idxs, ent_table, path_table):
    idxs = (ents_path_idxs.astype(jnp.int32)
            .reshape(NW, BPW, 3).transpose(0, 2, 1).reshape(NW, 3, NCH, CH))
    return _uni_model_sc(idxs, ent_table, path_table).reshape(B, 1, 1)


# speedup vs baseline: 1.4916x; 1.0049x over previous
import functools

import jax
import jax.numpy as jnp
from jax import lax
from jax.experimental import pallas as pl
from jax.experimental.pallas import tpu as pltpu
from jax.experimental.pallas import tpu_sc as plsc

B = 16384
D = 128
NC = 2
NS = 16
L = 16
NW = NC * NS
BPW = B // NW
CH = 128
NCH = BPW // CH
UD = 8
NACC = 4


def _sc_body(idx_hbm, ent_hbm, path_hbm, out_hbm,
             idx_v,
             pos0, neg0, path0, pos1, neg1, path1, out_v,
             sa0, sa1, sa2, sb0, sb1, sb2):
    w = lax.axis_index("s") * NC + lax.axis_index("c")
    base = w * BPW
    pltpu.sync_copy(idx_hbm.at[w], idx_v)

    bufs = ((pos0, neg0, path0, sa0, sa1, sa2),
            (pos1, neg1, path1, sb0, sb1, sb2))

    def issue(c, pos_b, neg_b, path_b, s0, s1, s2):
        return (pltpu.async_copy(ent_hbm.at[idx_v.at[1, c]], pos_b, s0),
                pltpu.async_copy(ent_hbm.at[idx_v.at[2, c]], neg_b, s1),
                pltpu.async_copy(path_hbm.at[idx_v.at[0, c]], path_b, s2))

    def compute(c, pos_b, neg_b, path_b, *_):
        lane = lax.iota(jnp.int32, L)

        def group_body(g, _):
            rows = lane + g * L

            def block_body(bb, accs):
                d0 = bb * UD
                accs = list(accs)
                for k in range(UD):
                    dsp = (lane + (d0 + k)) & (D - 1)
                    p = plsc.load_gather(pos_b, [rows, dsp])
                    n = plsc.load_gather(neg_b, [rows, dsp])
                    t = plsc.load_gather(path_b, [rows, dsp])
                    accs[k % NACC] = accs[k % NACC] + (p - n) * t
                return tuple(accs)

            zero = jnp.zeros((L,), jnp.float32)
            accs = lax.fori_loop(0, D // UD, block_body, (zero,) * NACC)
            acc = (accs[0] + accs[1]) + (accs[2] + accs[3])
            out_v[pl.ds(c * CH + g * L, L)] = acc
            return 0

        lax.fori_loop(0, CH // L, group_body, 0)

    pending = {0: issue(0, *bufs[0])}
    for c in range(NCH):
        if c + 1 < NCH:
            pending[c + 1] = issue(c + 1, *bufs[(c + 1) % 2])
        for desc in pending.pop(c):
            desc.wait()
        compute(c, *bufs[c % 2])

    pltpu.sync_copy(out_v, out_hbm.at[pl.ds(base, BPW)])


@functools.partial(
    pl.kernel,
    out_type=jax.ShapeDtypeStruct((B,), jnp.float32),
    mesh=plsc.VectorSubcoreMesh(core_axis_name="c", subcore_axis_name="s"),
    compiler_params=pltpu.CompilerParams(
        needs_layout_passes=False, disable_bounds_checks=True,
        disable_semaphore_checks=True),
    scratch_types=[
        pltpu.VMEM((3, NCH, CH), jnp.int32),
        pltpu.VMEM((CH, D), jnp.float32),
        pltpu.VMEM((CH, D), jnp.float32),
        pltpu.VMEM((CH, D), jnp.float32),
        pltpu.VMEM((CH, D), jnp.float32),
        pltpu.VMEM((CH, D), jnp.float32),
        pltpu.VMEM((CH, D), jnp.float32),
        pltpu.VMEM((BPW,), jnp.float32),
        pltpu.SemaphoreType.DMA,
        pltpu.SemaphoreType.DMA,
        pltpu.SemaphoreType.DMA,
        pltpu.SemaphoreType.DMA,
        pltpu.SemaphoreType.DMA,
        pltpu.SemaphoreType.DMA,
    ],
)
def _uni_model_sc(idx_hbm, ent_hbm, path_hbm, out_hbm, *rest):
    _sc_body(idx_hbm, ent_hbm, path_hbm, out_hbm, *rest)


@jax.jit
def kernel(ents_path_idxs, ent_table, path_table):
    idxs = (ents_path_idxs.astype(jnp.int32)
            .reshape(NW, BPW, 3).transpose(0, 2, 1).reshape(NW, 3, NCH, CH))
    return _uni_model_sc(idxs, ent_table, path_table).reshape(B, 1, 1)


# speedup vs baseline: 1.4939x; 1.0016x over previous
import functools

import jax
import jax.numpy as jnp
from jax import lax
from jax.experimental import pallas as pl
from jax.experimental.pallas import tpu as pltpu
from jax.experimental.pallas import tpu_sc as plsc

B = 16384
D = 128
NC = 2
NS = 16
L = 16
NW = NC * NS
BPW = B // NW
CH = 128
NCH = BPW // CH
UD = 8
NACC = 4


def _sc_body(idx_hbm, ent_hbm, path_hbm, out_hbm,
             idx_v, pos_b, neg_b, path_b, out_v, s0, s1, s2):
    w = lax.axis_index("s") * NC + lax.axis_index("c")
    base = w * BPW
    pltpu.sync_copy(idx_hbm.at[w], idx_v)

    def issue(c, slot):
        pltpu.async_copy(ent_hbm.at[idx_v.at[1, c]], pos_b.at[slot], s0.at[slot])
        pltpu.async_copy(ent_hbm.at[idx_v.at[2, c]], neg_b.at[slot], s1.at[slot])
        pltpu.async_copy(path_hbm.at[idx_v.at[0, c]], path_b.at[slot], s2.at[slot])

    issue(0, 0)
    issue(1, 1)
    lane = lax.iota(jnp.int32, L)

    def chunk_body(c, _):
        slot = c & 1
        slot_sp = jnp.full((L,), 0, jnp.int32) + slot
        pltpu.make_async_copy(ent_hbm.at[idx_v.at[1, c]], pos_b.at[slot], s0.at[slot]).wait()
        pltpu.make_async_copy(ent_hbm.at[idx_v.at[2, c]], neg_b.at[slot], s1.at[slot]).wait()
        pltpu.make_async_copy(path_hbm.at[idx_v.at[0, c]], path_b.at[slot], s2.at[slot]).wait()

        def group_body(g, _):
            rows = lane + g * L

            def block_body(bb, accs):
                d0 = bb * UD
                accs = list(accs)
                for k in range(UD):
                    dsp = (lane + (d0 + k)) & (D - 1)
                    p = plsc.load_gather(pos_b, [slot_sp, rows, dsp])
                    n = plsc.load_gather(neg_b, [slot_sp, rows, dsp])
                    t = plsc.load_gather(path_b, [slot_sp, rows, dsp])
                    accs[k % NACC] = accs[k % NACC] + (p - n) * t
                return tuple(accs)

            zero = jnp.zeros((L,), jnp.float32)
            accs = lax.fori_loop(0, D // UD, block_body, (zero,) * NACC)
            acc = (accs[0] + accs[1]) + (accs[2] + accs[3])
            out_v[pl.ds(c * CH + g * L, L)] = acc
            return 0

        lax.fori_loop(0, CH // L, group_body, 0)

        @pl.when(c < NCH - 2)
        def _():
            issue(c + 2, slot)

        return 0

    lax.fori_loop(0, NCH, chunk_body, 0)

    pltpu.sync_copy(out_v, out_hbm.at[pl.ds(base, BPW)])


@functools.partial(
    pl.kernel,
    out_type=jax.ShapeDtypeStruct((B,), jnp.float32),
    mesh=plsc.VectorSubcoreMesh(core_axis_name="c", subcore_axis_name="s"),
    compiler_params=pltpu.CompilerParams(
        needs_layout_passes=False, disable_bounds_checks=True,
        disable_semaphore_checks=True),
    scratch_types=[
        pltpu.VMEM((3, NCH, CH), jnp.int32),
        pltpu.VMEM((2, CH, D), jnp.float32),
        pltpu.VMEM((2, CH, D), jnp.float32),
        pltpu.VMEM((2, CH, D), jnp.float32),
        pltpu.VMEM((BPW,), jnp.float32),
        pltpu.SemaphoreType.DMA((2,)),
        pltpu.SemaphoreType.DMA((2,)),
        pltpu.SemaphoreType.DMA((2,)),
    ],
)
def _uni_model_sc(idx_hbm, ent_hbm, path_hbm, out_hbm, *rest):
    _sc_body(idx_hbm, ent_hbm, path_hbm, out_hbm, *rest)


@jax.jit
def kernel(ents_path_idxs, ent_table, path_table):
    idxs = (ents_path_idxs.astype(jnp.int32)
            .reshape(NW, BPW, 3).transpose(0, 2, 1).reshape(NW, 3, NCH, CH))
    return _uni_model_sc(idxs, ent_table, path_table).reshape(B, 1, 1)


# speedup vs baseline: 1.5047x; 1.0072x over previous
import functools

import jax
import jax.numpy as jnp
from jax import lax
from jax.experimental import pallas as pl
from jax.experimental.pallas import tpu as pltpu
from jax.experimental.pallas import tpu_sc as plsc

B = 16384
D = 128
NC = 2
NS = 16
L = 16
NW = NC * NS
BPW = B // NW
CH = 128
NCH = BPW // CH
UD = 16
NACC = 4


def _sc_body(idx_hbm, ent_hbm, path_hbm, out_hbm,
             idx_v, pos_b, neg_b, path_b, out_v, s0, s1, s2):
    w = lax.axis_index("s") * NC + lax.axis_index("c")
    base = w * BPW
    pltpu.sync_copy(idx_hbm.at[w], idx_v)

    def issue(c, slot):
        pltpu.async_copy(ent_hbm.at[idx_v.at[1, c]], pos_b.at[slot], s0.at[slot])
        pltpu.async_copy(ent_hbm.at[idx_v.at[2, c]], neg_b.at[slot], s1.at[slot])
        pltpu.async_copy(path_hbm.at[idx_v.at[0, c]], path_b.at[slot], s2.at[slot])

    issue(0, 0)
    issue(1, 1)
    lane = lax.iota(jnp.int32, L)

    def chunk_body(c, _):
        slot = c & 1
        slot_sp = jnp.full((L,), 0, jnp.int32) + slot
        pltpu.make_async_copy(ent_hbm.at[idx_v.at[1, c]], pos_b.at[slot], s0.at[slot]).wait()
        pltpu.make_async_copy(ent_hbm.at[idx_v.at[2, c]], neg_b.at[slot], s1.at[slot]).wait()
        pltpu.make_async_copy(path_hbm.at[idx_v.at[0, c]], path_b.at[slot], s2.at[slot]).wait()

        def group_body(g, _):
            rows = lane + g * L

            def block_body(bb, accs):
                d0 = bb * UD
                accs = list(accs)
                for k in range(UD):
                    dsp = (lane + (d0 + k)) & (D - 1)
                    p = plsc.load_gather(pos_b, [slot_sp, rows, dsp])
                    n = plsc.load_gather(neg_b, [slot_sp, rows, dsp])
                    t = plsc.load_gather(path_b, [slot_sp, rows, dsp])
                    accs[k % NACC] = accs[k % NACC] + (p - n) * t
                return tuple(accs)

            zero = jnp.zeros((L,), jnp.float32)
            accs = lax.fori_loop(0, D // UD, block_body, (zero,) * NACC)
            acc = (accs[0] + accs[1]) + (accs[2] + accs[3])
            out_v[pl.ds(c * CH + g * L, L)] = acc
            return 0

        lax.fori_loop(0, CH // L, group_body, 0)

        @pl.when(c < NCH - 2)
        def _():
            issue(c + 2, slot)

        return 0

    lax.fori_loop(0, NCH, chunk_body, 0)

    pltpu.sync_copy(out_v, out_hbm.at[pl.ds(base, BPW)])


@functools.partial(
    pl.kernel,
    out_type=jax.ShapeDtypeStruct((B,), jnp.float32),
    mesh=plsc.VectorSubcoreMesh(core_axis_name="c", subcore_axis_name="s"),
    compiler_params=pltpu.CompilerParams(
        needs_layout_passes=False, disable_bounds_checks=True,
        disable_semaphore_checks=True),
    scratch_types=[
        pltpu.VMEM((3, NCH, CH), jnp.int32),
        pltpu.VMEM((2, CH, D), jnp.float32),
        pltpu.VMEM((2, CH, D), jnp.float32),
        pltpu.VMEM((2, CH, D), jnp.float32),
        pltpu.VMEM((BPW,), jnp.float32),
        pltpu.SemaphoreType.DMA((2,)),
        pltpu.SemaphoreType.DMA((2,)),
        pltpu.SemaphoreType.DMA((2,)),
    ],
)
def _uni_model_sc(idx_hbm, ent_hbm, path_hbm, out_hbm, *rest):
    _sc_body(idx_hbm, ent_hbm, path_hbm, out_hbm, *rest)


@jax.jit
def kernel(ents_path_idxs, ent_table, path_table):
    idxs = (ents_path_idxs.astype(jnp.int32)
            .reshape(NW, BPW, 3).transpose(0, 2, 1).reshape(NW, 3, NCH, CH))
    return _uni_model_sc(idxs, ent_table, path_table).reshape(B, 1, 1)
